# baseline (speedup 1.0000x reference)
import functools

import jax
import jax.numpy as jnp
from jax import lax
from jax.experimental import pallas as pl
from jax.experimental.pallas import tpu as pltpu
from jax.experimental.pallas import tpu_sc as plsc

_N, _H, _W = 8, 512, 512
_HW = _H * _W
_NEG_RATIO = 3.0
_EPS = 1e-08

_NC = 2
_NS = 16
_SPL = _N // _NC
_CHUNK = _HW // _NS
_NV = _CHUNK // 16


def _tc_loss_kernel(pred_ref, gt_ref, mask_ref, vbits_ref, scal_ref):
    p = pred_ref[0]
    g = gt_ref[0]
    m = mask_ref[0]

    pos = g * m
    neg = (1.0 - g) * m
    pos_count = jnp.sum(pos)
    neg_count_raw = jnp.sum(neg)
    neg_count = jnp.minimum(neg_count_raw, pos_count * _NEG_RATIO)
    k_f = jnp.floor(neg_count)

    log_p = jnp.maximum(jnp.log(p), -100.0)
    log_1mp = jnp.maximum(jnp.log(1.0 - p), -100.0)
    loss = -(g * log_p + (1.0 - g) * log_1mp)
    pos_loss_sum = jnp.sum(loss * pos)

    v = loss * neg
    vbits_ref[0] = lax.bitcast_convert_type(v, jnp.int32)
    scal_ref[:, :, :] = jnp.stack(
        [pos_count, neg_count, pos_loss_sum, k_f]
    ).reshape(1, 1, 4)


def _sc_select_kernel(vbits_hbm, k_hbm, out_hbm, data_v, kv, hist, chist,
                      tmp4, sumv, fzero, shist):
    core = lax.axis_index("c")
    sub = lax.axis_index("s")

    lanes = lax.iota(jnp.int32, 16)
    lane_base = lanes * 256
    ones_i = jnp.ones((16,), jnp.int32)
    zeros_i = jnp.zeros((16,), jnp.int32)
    zeros_f = jnp.zeros((16,), jnp.float32)

    pltpu.sync_copy(k_hbm, kv)
    for j in range(_SPL):
        n = 2 * j + core
        pltpu.sync_copy(
            vbits_hbm.at[n, pl.ds(sub * _CHUNK, _CHUNK)], data_v.at[j]
        )

    def _z(i, _):
        hist[pl.ds(i * 16, 16)] = zeros_i
        return 0
    lax.fori_loop(0, 256, _z, 0)
    fzero[0, :] = zeros_f

    kvec = kv[...]
    ks = [
        jnp.sum(jnp.where(lanes == 2 * j + core, kvec, 0))
        for j in range(_SPL)
    ]
    k_cur = list(ks)
    prefix = [jnp.int32(0)] * _SPL
    cnt_above = [jnp.int32(0)] * _SPL

    shifts = (23, 15, 7)
    for level in range(3):
        sh = shifts[level]
        for j in range(_SPL):
            pref_j = prefix[j]

            def _histbody(i, _, j=j, pref_j=pref_j, sh=sh, level=level):
                w = data_v[j, pl.ds(i * 16, 16)]
                d = jnp.bitwise_and(lax.shift_right_logical(w, sh), 255)
                idx = lane_base + d
                if level == 0:
                    plsc.addupdate_scatter(hist, [idx], ones_i)
                else:
                    msk = lax.shift_right_logical(w, sh + 8) == pref_j
                    plsc.addupdate_scatter(hist, [idx], ones_i, mask=msk)
                return 0

            lax.fori_loop(0, _NV, _histbody, 0)

            def _collapse(c, _, j=j):
                acc = zeros_i
                for r in range(16):
                    acc = acc + hist[pl.ds(r * 256 + c * 16, 16)]
                    hist[pl.ds(r * 256 + c * 16, 16)] = zeros_i
                chist[j * 16 + c, :] = acc
                return 0

            lax.fori_loop(0, 16, _collapse, 0)

        base = core * 1024
        pltpu.sync_copy(chist, shist.at[pl.ds(base + sub * 64, 64)])
        plsc.subcore_barrier()
        accs = [zeros_i, zeros_i, zeros_i, zeros_i]
        for t in range(16):
            pltpu.sync_copy(
                shist.at[pl.ds(base + t * 64 + sub * 4, 4)], tmp4
            )
            for r in range(4):
                accs[r] = accs[r] + tmp4[r, :]
        for r in range(4):
            tmp4[r, :] = accs[r]
        pltpu.sync_copy(tmp4, shist.at[pl.ds(base + sub * 4, 4)])
        plsc.subcore_barrier()
        pltpu.sync_copy(shist.at[pl.ds(base, 64)], chist)
        plsc.subcore_barrier()

        for j in range(_SPL):
            kj = k_cur[j]

            def _scan(c, carry, j=j, kj=kj):
                acc, bstar, s_at = carry
                cc = 15 - c
                h = chist[j * 16 + cc, :]
                t_inc = lax.rev(jnp.cumsum(lax.rev(h, (0,))), (0,))
                s_i = acc + t_inc - h
                sel = jnp.logical_and(s_i < kj, s_i + h >= kj)
                binidx = cc * 16 + lanes
                bstar = jnp.maximum(
                    bstar, jnp.max(jnp.where(sel, binidx, -1))
                )
                s_at = s_at + jnp.sum(jnp.where(sel, s_i, 0))
                acc = acc + jnp.sum(h)
                return acc, bstar, s_at

            acc0 = (jnp.int32(0), jnp.int32(-1), jnp.int32(0))
            _, bstar, s_at = lax.fori_loop(0, 16, _scan, acc0)
            prefix[j] = jnp.bitwise_or(prefix[j] * 256, jnp.maximum(bstar, 0))
            cnt_above[j] = cnt_above[j] + s_at
            k_cur[j] = kj - s_at

    sums_vec = zeros_f
    for j in range(_SPL):
        p_hi = prefix[j] * 128 + 128

        def _sum(i, a, j=j, p_hi=p_hi):
            w = data_v[j, pl.ds(i * 16, 16)]
            vals = plsc.bitcast(w, jnp.float32)
            return a + jnp.where(w >= p_hi, vals, 0.0)

        a = lax.fori_loop(0, _NV, _sum, zeros_f)
        sums_vec = sums_vec + jnp.where(lanes == j, jnp.sum(a), 0.0)

    sumv[0, :] = sums_vec
    pltpu.sync_copy(sumv.at[pl.ds(0, 1)], out_hbm.at[core, pl.ds(sub, 1)])

    @pl.when(sub == 0)
    def _():
        t_bits_vec = zeros_i
        rem_vec_i = zeros_i
        for j in range(_SPL):
            lane_j = lanes == j
            t_bits_vec = t_bits_vec + jnp.where(lane_j, prefix[j] * 128, 0)
            rem_vec_i = rem_vec_i + jnp.where(lane_j, ks[j] - cnt_above[j], 0)
        fzero[0, :] = plsc.bitcast(t_bits_vec, jnp.float32)
        pltpu.sync_copy(fzero, out_hbm.at[core, pl.ds(16, 1)])
        fzero[0, :] = rem_vec_i.astype(jnp.float32)
        pltpu.sync_copy(fzero, out_hbm.at[core, pl.ds(17, 1)])


@functools.partial(
    pl.kernel,
    out_type=jax.ShapeDtypeStruct((_NC, 18, 16), jnp.float32),
    mesh=plsc.VectorSubcoreMesh(core_axis_name="c", subcore_axis_name="s"),
    compiler_params=pltpu.CompilerParams(needs_layout_passes=False),
    scratch_types=[
        pltpu.VMEM((_SPL, _CHUNK), jnp.int32),
        pltpu.VMEM((16,), jnp.int32),
        pltpu.VMEM((16 * 256,), jnp.int32),
        pltpu.VMEM((_SPL * 16, 16), jnp.int32),
        pltpu.VMEM((4, 16), jnp.int32),
        pltpu.VMEM((16, 16), jnp.float32),
        pltpu.VMEM((1, 16), jnp.float32),
        pltpu.VMEM_SHARED((_NC * 16 * 64, 16), jnp.int32),
    ],
)
def _sc_select(vbits, kvec, out, *scratch):
    _sc_select_kernel(vbits, kvec, out, *scratch)


@jax.jit
def kernel(pred, gt, mask):
    p = pred.reshape(_N, _H, _W)
    vbits, scal = pl.pallas_call(
        _tc_loss_kernel,
        grid=(_N,),
        in_specs=[
            pl.BlockSpec((1, _H, _W), lambda i: (i, 0, 0)),
            pl.BlockSpec((1, _H, _W), lambda i: (i, 0, 0)),
            pl.BlockSpec((1, _H, _W), lambda i: (i, 0, 0)),
        ],
        out_specs=[
            pl.BlockSpec((1, _H, _W), lambda i: (i, 0, 0)),
            pl.BlockSpec((1, 1, 4), lambda i: (i, 0, 0)),
        ],
        out_shape=[
            jax.ShapeDtypeStruct((_N, _H, _W), jnp.int32),
            jax.ShapeDtypeStruct((_N, 1, 4), jnp.float32),
        ],
    )(p, gt, mask)

    pos_count = scal[:, 0, 0]
    neg_count = scal[:, 0, 1]
    pos_loss_sum = scal[:, 0, 2]
    kvec = jnp.zeros((16,), jnp.int32).at[:_N].set(scal[:, 0, 3].astype(jnp.int32))

    out = _sc_select(vbits.reshape(_N, _HW), kvec)
    tops_all = out[:, :16, :].sum(axis=1) + out[:, 17, :] * out[:, 16, :]
    top_neg = tops_all[jnp.arange(_N) % _NC, jnp.arange(_N) // _NC]
    top_neg = jnp.where(scal[:, 0, 3] > 0, top_neg, 0.0)

    per_sample = (pos_loss_sum + top_neg) / (pos_count + neg_count + _EPS)
    return jnp.sum(per_sample) / _N


# speedup vs baseline: 1.1223x; 1.1223x over previous
import functools

import jax
import jax.numpy as jnp
from jax import lax
from jax.experimental import pallas as pl
from jax.experimental.pallas import tpu as pltpu
from jax.experimental.pallas import tpu_sc as plsc

_N, _H, _W = 8, 512, 512
_HW = _H * _W
_NEG_RATIO = 3.0
_EPS = 1e-08

_NC = 2
_NS = 16
_SPL = _N // _NC
_CHUNK = _HW // _NS
_NV = _CHUNK // 16


def _tc_loss_kernel(pred_ref, gt_ref, mask_ref, vbits_ref, scal_ref):
    p = pred_ref[0]
    g = gt_ref[0]
    m = mask_ref[0]

    pos = g * m
    neg = (1.0 - g) * m
    pos_count = jnp.sum(pos)
    neg_count_raw = jnp.sum(neg)
    neg_count = jnp.minimum(neg_count_raw, pos_count * _NEG_RATIO)
    k_f = jnp.floor(neg_count)

    log_p = jnp.maximum(jnp.log(p), -100.0)
    log_1mp = jnp.maximum(jnp.log(1.0 - p), -100.0)
    loss = -(g * log_p + (1.0 - g) * log_1mp)
    pos_loss_sum = jnp.sum(loss * pos)

    v = loss * neg
    vbits_ref[0, 0] = lax.bitcast_convert_type(v, jnp.int32).reshape(_HW)
    scal_ref[:, :, :] = jnp.stack(
        [pos_count, neg_count, pos_loss_sum, k_f]
    ).reshape(1, 1, 4)


def _sc_select_kernel(vbits_hbm, k_hbm, out_hbm, data_v, kv, hist, chist,
                      tmp4, sumv, fzero, shist):
    core = lax.axis_index("c")
    sub = lax.axis_index("s")

    lanes = lax.iota(jnp.int32, 16)
    lane_base = lanes * 256
    ones_i = jnp.ones((16,), jnp.int32)
    zeros_i = jnp.zeros((16,), jnp.int32)
    zeros_f = jnp.zeros((16,), jnp.float32)

    pltpu.sync_copy(k_hbm, kv)
    for j in range(_SPL):
        n = 2 * j + core
        pltpu.sync_copy(
            vbits_hbm.at[n, pl.ds(sub * _CHUNK, _CHUNK)], data_v.at[j]
        )

    def _z(i, _):
        hist[pl.ds(i * 16, 16)] = zeros_i
        return 0
    lax.fori_loop(0, 256, _z, 0)
    fzero[0, :] = zeros_f

    kvec = kv[...]
    ks = [
        jnp.sum(jnp.where(lanes == 2 * j + core, kvec, 0))
        for j in range(_SPL)
    ]
    k_cur = list(ks)
    prefix = [jnp.int32(0)] * _SPL
    cnt_above = [jnp.int32(0)] * _SPL

    shifts = (23, 15, 7)
    for level in range(3):
        sh = shifts[level]
        for j in range(_SPL):
            pref_j = prefix[j]

            def _histbody(i, _, j=j, pref_j=pref_j, sh=sh, level=level):
                for u in range(8):
                    w = data_v[j, pl.ds((i * 8 + u) * 16, 16)]
                    d = jnp.bitwise_and(lax.shift_right_logical(w, sh), 255)
                    idx = lane_base + d
                    if level == 0:
                        plsc.addupdate_scatter(hist, [idx], ones_i)
                    else:
                        msk = lax.shift_right_logical(w, sh + 8) == pref_j
                        plsc.addupdate_scatter(hist, [idx], ones_i, mask=msk)
                return 0

            lax.fori_loop(0, _NV // 8, _histbody, 0)

            def _collapse(c, _, j=j):
                acc = zeros_i
                for r in range(16):
                    acc = acc + hist[pl.ds(r * 256 + c * 16, 16)]
                    hist[pl.ds(r * 256 + c * 16, 16)] = zeros_i
                chist[j * 16 + c, :] = acc
                return 0

            lax.fori_loop(0, 16, _collapse, 0)

        base = core * 1024
        pltpu.sync_copy(chist, shist.at[pl.ds(base + sub * 64, 64)])
        plsc.subcore_barrier()
        accs = [zeros_i, zeros_i, zeros_i, zeros_i]
        for t in range(16):
            pltpu.sync_copy(
                shist.at[pl.ds(base + t * 64 + sub * 4, 4)], tmp4
            )
            for r in range(4):
                accs[r] = accs[r] + tmp4[r, :]
        for r in range(4):
            tmp4[r, :] = accs[r]
        pltpu.sync_copy(tmp4, shist.at[pl.ds(base + sub * 4, 4)])
        plsc.subcore_barrier()
        pltpu.sync_copy(shist.at[pl.ds(base, 64)], chist)
        plsc.subcore_barrier()

        for j in range(_SPL):
            kj = k_cur[j]

            def _scan(c, carry, j=j, kj=kj):
                acc, bstar, s_at = carry
                cc = 15 - c
                h = chist[j * 16 + cc, :]
                t_inc = lax.rev(jnp.cumsum(lax.rev(h, (0,))), (0,))
                s_i = acc + t_inc - h
                sel = jnp.logical_and(s_i < kj, s_i + h >= kj)
                binidx = cc * 16 + lanes
                bstar = jnp.maximum(
                    bstar, jnp.max(jnp.where(sel, binidx, -1))
                )
                s_at = s_at + jnp.sum(jnp.where(sel, s_i, 0))
                acc = acc + jnp.sum(h)
                return acc, bstar, s_at

            acc0 = (jnp.int32(0), jnp.int32(-1), jnp.int32(0))
            _, bstar, s_at = lax.fori_loop(0, 16, _scan, acc0)
            prefix[j] = jnp.bitwise_or(prefix[j] * 256, jnp.maximum(bstar, 0))
            cnt_above[j] = cnt_above[j] + s_at
            k_cur[j] = kj - s_at

    sums_vec = zeros_f
    for j in range(_SPL):
        p_hi = prefix[j] * 128 + 128

        def _sum(i, a, j=j, p_hi=p_hi):
            for u in range(8):
                w = data_v[j, pl.ds((i * 8 + u) * 16, 16)]
                vals = plsc.bitcast(w, jnp.float32)
                a = a + jnp.where(w >= p_hi, vals, 0.0)
            return a

        a = lax.fori_loop(0, _NV // 8, _sum, zeros_f)
        sums_vec = sums_vec + jnp.where(lanes == j, jnp.sum(a), 0.0)

    sumv[0, :] = sums_vec
    pltpu.sync_copy(sumv.at[pl.ds(0, 1)], out_hbm.at[core, pl.ds(sub, 1)])

    @pl.when(sub == 0)
    def _():
        t_bits_vec = zeros_i
        rem_vec_i = zeros_i
        for j in range(_SPL):
            lane_j = lanes == j
            t_bits_vec = t_bits_vec + jnp.where(lane_j, prefix[j] * 128, 0)
            rem_vec_i = rem_vec_i + jnp.where(lane_j, ks[j] - cnt_above[j], 0)
        fzero[0, :] = plsc.bitcast(t_bits_vec, jnp.float32)
        pltpu.sync_copy(fzero, out_hbm.at[core, pl.ds(16, 1)])
        fzero[0, :] = rem_vec_i.astype(jnp.float32)
        pltpu.sync_copy(fzero, out_hbm.at[core, pl.ds(17, 1)])


@functools.partial(
    pl.kernel,
    out_type=jax.ShapeDtypeStruct((_NC, 18, 16), jnp.float32),
    mesh=plsc.VectorSubcoreMesh(core_axis_name="c", subcore_axis_name="s"),
    compiler_params=pltpu.CompilerParams(needs_layout_passes=False),
    scratch_types=[
        pltpu.VMEM((_SPL, _CHUNK), jnp.int32),
        pltpu.VMEM((16,), jnp.int32),
        pltpu.VMEM((16 * 256,), jnp.int32),
        pltpu.VMEM((_SPL * 16, 16), jnp.int32),
        pltpu.VMEM((4, 16), jnp.int32),
        pltpu.VMEM((16, 16), jnp.float32),
        pltpu.VMEM((1, 16), jnp.float32),
        pltpu.VMEM_SHARED((_NC * 16 * 64, 16), jnp.int32),
    ],
)
def _sc_select(vbits, kvec, out, *scratch):
    _sc_select_kernel(vbits, kvec, out, *scratch)


@jax.jit
def kernel(pred, gt, mask):
    p = pred.reshape(_N, _H, _W)
    vbits, scal = pl.pallas_call(
        _tc_loss_kernel,
        grid=(_N,),
        in_specs=[
            pl.BlockSpec((1, _H, _W), lambda i: (i, 0, 0)),
            pl.BlockSpec((1, _H, _W), lambda i: (i, 0, 0)),
            pl.BlockSpec((1, _H, _W), lambda i: (i, 0, 0)),
        ],
        out_specs=[
            pl.BlockSpec((1, 1, _HW), lambda i: (i, 0, 0)),
            pl.BlockSpec((1, 1, 4), lambda i: (i, 0, 0)),
        ],
        out_shape=[
            jax.ShapeDtypeStruct((_N, 1, _HW), jnp.int32),
            jax.ShapeDtypeStruct((_N, 1, 4), jnp.float32),
        ],
    )(p, gt, mask)

    pos_count = scal[:, 0, 0]
    neg_count = scal[:, 0, 1]
    pos_loss_sum = scal[:, 0, 2]
    kvec = jnp.zeros((16,), jnp.int32).at[:_N].set(scal[:, 0, 3].astype(jnp.int32))

    out = _sc_select(vbits.reshape(_N, _HW), kvec)
    tops_all = out[:, :16, :].sum(axis=1) + out[:, 17, :] * out[:, 16, :]
    top_neg = tops_all[jnp.arange(_N) % _NC, jnp.arange(_N) // _NC]
    top_neg = jnp.where(scal[:, 0, 3] > 0, top_neg, 0.0)

    per_sample = (pos_loss_sum + top_neg) / (pos_count + neg_count + _EPS)
    return jnp.sum(per_sample) / _N


# speedup vs baseline: 1.3535x; 1.2060x over previous
import functools

import jax
import jax.numpy as jnp
from jax import lax
from jax.experimental import pallas as pl
from jax.experimental.pallas import tpu as pltpu
from jax.experimental.pallas import tpu_sc as plsc

_N, _H, _W = 8, 512, 512
_HW = _H * _W
_NEG_RATIO = 3.0
_EPS = 1e-08

_NC = 2
_NS = 16
_SPL = _N // _NC
_CHUNK = _HW // _NS
_NV = _CHUNK // 16

_LB = 9
_BINS = 1 << _LB
_NCH = _BINS // 16
_ROWS = _SPL * _NCH


def _tc_loss_kernel(pred_ref, gt_ref, mask_ref, vbits_ref, scal_ref):
    p = pred_ref[0]
    g = gt_ref[0]
    m = mask_ref[0]

    pos = g * m
    neg = (1.0 - g) * m
    pos_count = jnp.sum(pos)
    neg_count_raw = jnp.sum(neg)
    neg_count = jnp.minimum(neg_count_raw, pos_count * _NEG_RATIO)
    k_f = jnp.floor(neg_count)

    log_p = jnp.maximum(jnp.log(p), -100.0)
    log_1mp = jnp.maximum(jnp.log(1.0 - p), -100.0)
    loss = -(g * log_p + (1.0 - g) * log_1mp)
    pos_loss_sum = jnp.sum(loss * pos)

    v = loss * neg
    vbits_ref[0, 0] = lax.bitcast_convert_type(v, jnp.int32).reshape(_HW)
    scal_ref[:, :, :] = jnp.stack(
        [pos_count, neg_count, pos_loss_sum, k_f]
    ).reshape(1, 1, 4)


def _sc_select_kernel(vbits_hbm, k_hbm, out_hbm, data_v, kv, hist, chist,
                      tmp8, sumv, fzero, shist):
    core = lax.axis_index("c")
    sub = lax.axis_index("s")

    lanes = lax.iota(jnp.int32, 16)
    lane_base = lanes * _BINS
    ones_i = jnp.ones((16,), jnp.int32)
    zeros_i = jnp.zeros((16,), jnp.int32)
    zeros_f = jnp.zeros((16,), jnp.float32)

    pltpu.sync_copy(k_hbm, kv)
    for j in range(_SPL):
        n = 2 * j + core
        pltpu.sync_copy(
            vbits_hbm.at[n, pl.ds(sub * _CHUNK, _CHUNK)], data_v.at[j]
        )

    def _z(i, _):
        hist[pl.ds(i * 16, 16)] = zeros_i
        return 0
    lax.fori_loop(0, 16 * _BINS // 16, _z, 0)
    fzero[0, :] = zeros_f

    kvec = kv[...]
    ks = [
        jnp.sum(jnp.where(lanes == 2 * j + core, kvec, 0))
        for j in range(_SPL)
    ]
    k_cur = list(ks)
    prefix = [jnp.int32(0)] * _SPL
    cnt_above = [jnp.int32(0)] * _SPL

    shifts = (31 - _LB, 31 - 2 * _LB)
    for level in range(2):
        sh = shifts[level]
        for j in range(_SPL):
            pref_j = prefix[j]

            def _histbody(i, _, j=j, pref_j=pref_j, sh=sh, level=level):
                for u in range(8):
                    w = data_v[j, pl.ds((i * 8 + u) * 16, 16)]
                    d = jnp.bitwise_and(
                        lax.shift_right_logical(w, sh), _BINS - 1
                    )
                    idx = lane_base + d
                    if level == 0:
                        plsc.addupdate_scatter(hist, [idx], ones_i)
                    else:
                        msk = lax.shift_right_logical(w, sh + _LB) == pref_j
                        plsc.addupdate_scatter(hist, [idx], ones_i, mask=msk)
                return 0

            lax.fori_loop(0, _NV // 8, _histbody, 0)

            def _collapse(c, _, j=j):
                acc = zeros_i
                for r in range(16):
                    acc = acc + hist[pl.ds(r * _BINS + c * 16, 16)]
                    hist[pl.ds(r * _BINS + c * 16, 16)] = zeros_i
                chist[j * _NCH + c, :] = acc
                return 0

            lax.fori_loop(0, _NCH, _collapse, 0)

        base = core * _NS * _ROWS
        stripe = _ROWS // 16
        pltpu.sync_copy(chist, shist.at[pl.ds(base + sub * _ROWS, _ROWS)])
        plsc.subcore_barrier()
        accs = [zeros_i] * stripe
        for t in range(16):
            pltpu.sync_copy(
                shist.at[pl.ds(base + t * _ROWS + sub * stripe, stripe)], tmp8
            )
            for r in range(stripe):
                accs[r] = accs[r] + tmp8[r, :]
        for r in range(stripe):
            tmp8[r, :] = accs[r]
        pltpu.sync_copy(tmp8, shist.at[pl.ds(base + sub * stripe, stripe)])
        plsc.subcore_barrier()
        pltpu.sync_copy(shist.at[pl.ds(base, _ROWS)], chist)
        plsc.subcore_barrier()

        for j in range(_SPL):
            kj = k_cur[j]

            def _scan(c, carry, j=j, kj=kj):
                acc, bstar, s_at = carry
                cc = _NCH - 1 - c
                h = chist[j * _NCH + cc, :]
                t_inc = lax.rev(jnp.cumsum(lax.rev(h, (0,))), (0,))
                s_i = acc + t_inc - h
                sel = jnp.logical_and(s_i < kj, s_i + h >= kj)
                binidx = cc * 16 + lanes
                bstar = jnp.maximum(bstar, jnp.max(jnp.where(sel, binidx, -1)))
                s_at = s_at + jnp.sum(jnp.where(sel, s_i, 0))
                acc = acc + jnp.sum(h)
                return acc, bstar, s_at

            acc0 = (jnp.int32(0), jnp.int32(-1), jnp.int32(0))
            _, bstar, s_at = lax.fori_loop(0, _NCH, _scan, acc0)
            prefix[j] = jnp.bitwise_or(prefix[j] * _BINS, jnp.maximum(bstar, 0))
            cnt_above[j] = cnt_above[j] + s_at
            k_cur[j] = kj - s_at

    low = 1 << (31 - 2 * _LB)
    sums_vec = zeros_f
    for j in range(_SPL):
        p_hi = prefix[j] * low + low

        def _sum(i, accs, j=j, p_hi=p_hi):
            res = []
            for u in range(8):
                w = data_v[j, pl.ds((i * 8 + u) * 16, 16)]
                vals = plsc.bitcast(w, jnp.float32)
                res.append(accs[u] + jnp.where(w >= p_hi, vals, 0.0))
            return tuple(res)

        accs = lax.fori_loop(0, _NV // 8, _sum, (zeros_f,) * 8)
        tot = accs[0]
        for u in range(1, 8):
            tot = tot + accs[u]
        sums_vec = sums_vec + jnp.where(lanes == j, jnp.sum(tot), 0.0)

    sumv[0, :] = sums_vec
    pltpu.sync_copy(sumv, out_hbm.at[core, pl.ds(sub, 1)])

    @pl.when(sub == 0)
    def _():
        t_bits_vec = zeros_i
        rem_vec_i = zeros_i
        for j in range(_SPL):
            lane_j = lanes == j
            t_bits_vec = t_bits_vec + jnp.where(lane_j, prefix[j] * low, 0)
            rem_vec_i = rem_vec_i + jnp.where(lane_j, ks[j] - cnt_above[j], 0)
        fzero[0, :] = plsc.bitcast(t_bits_vec, jnp.float32)
        pltpu.sync_copy(fzero, out_hbm.at[core, pl.ds(16, 1)])
        fzero[0, :] = rem_vec_i.astype(jnp.float32)
        pltpu.sync_copy(fzero, out_hbm.at[core, pl.ds(17, 1)])


@functools.partial(
    pl.kernel,
    out_type=jax.ShapeDtypeStruct((_NC, 18, 16), jnp.float32),
    mesh=plsc.VectorSubcoreMesh(core_axis_name="c", subcore_axis_name="s"),
    compiler_params=pltpu.CompilerParams(needs_layout_passes=False),
    scratch_types=[
        pltpu.VMEM((_SPL, _CHUNK), jnp.int32),
        pltpu.VMEM((16,), jnp.int32),
        pltpu.VMEM((16 * _BINS,), jnp.int32),
        pltpu.VMEM((_ROWS, 16), jnp.int32),
        pltpu.VMEM((_ROWS // 16, 16), jnp.int32),
        pltpu.VMEM((1, 16), jnp.float32),
        pltpu.VMEM((1, 16), jnp.float32),
        pltpu.VMEM_SHARED((_NC * _NS * _ROWS, 16), jnp.int32),
    ],
)
def _sc_select(vbits, kvec, out, *scratch):
    _sc_select_kernel(vbits, kvec, out, *scratch)


@jax.jit
def kernel(pred, gt, mask):
    p = pred.reshape(_N, _H, _W)
    vbits, scal = pl.pallas_call(
        _tc_loss_kernel,
        grid=(_N,),
        in_specs=[
            pl.BlockSpec((1, _H, _W), lambda i: (i, 0, 0)),
            pl.BlockSpec((1, _H, _W), lambda i: (i, 0, 0)),
            pl.BlockSpec((1, _H, _W), lambda i: (i, 0, 0)),
        ],
        out_specs=[
            pl.BlockSpec((1, 1, _HW), lambda i: (i, 0, 0)),
            pl.BlockSpec((1, 1, 4), lambda i: (i, 0, 0)),
        ],
        out_shape=[
            jax.ShapeDtypeStruct((_N, 1, _HW), jnp.int32),
            jax.ShapeDtypeStruct((_N, 1, 4), jnp.float32),
        ],
    )(p, gt, mask)

    pos_count = scal[:, 0, 0]
    neg_count = scal[:, 0, 1]
    pos_loss_sum = scal[:, 0, 2]
    kvec = jnp.zeros((16,), jnp.int32).at[:_N].set(scal[:, 0, 3].astype(jnp.int32))

    out = _sc_select(vbits.reshape(_N, _HW), kvec)
    tops_all = out[:, :16, :].sum(axis=1) + out[:, 17, :] * out[:, 16, :]
    top_neg = tops_all[jnp.arange(_N) % _NC, jnp.arange(_N) // _NC]
    top_neg = jnp.where(scal[:, 0, 3] > 0, top_neg, 0.0)

    per_sample = (pos_loss_sum + top_neg) / (pos_count + neg_count + _EPS)
    return jnp.sum(per_sample) / _N
